# baseline (speedup 1.0000x reference)
import functools

import jax
import jax.numpy as jnp
from jax import lax
from jax.experimental import pallas as pl
from jax.experimental.pallas import tpu as pltpu
from jax.experimental.pallas import tpu_sc as plsc

HID = 128
NUM_E = 10000
NUM_EDGES = 320000
NUM_REL = 200

NC = 2
NS = 16
NT = NC * NS
E_PER_TILE = NUM_EDGES // NT
CHUNK = 40
N_CHUNKS = E_PER_TILE // CHUNK
NSLOT = 4
ZBLK = 40
N_ROW_BLOCKS = NUM_E // ZBLK
BLOCKS_PER_TILE = (N_ROW_BLOCKS + NS - 1) // NS
LANES = 16
VPR = HID // LANES


@functools.partial(
    pl.kernel,
    out_type=jax.ShapeDtypeStruct((NC, NUM_E, HID), jnp.float32),
    mesh=plsc.VectorSubcoreMesh(core_axis_name="c", subcore_axis_name="s"),
    compiler_params=pltpu.CompilerParams(needs_layout_passes=False),
    scratch_types=[
        pltpu.VMEM((NSLOT, 2, CHUNK), jnp.int32),
        pltpu.VMEM((NSLOT, CHUNK), jnp.float32),
        pltpu.VMEM((CHUNK, HID), jnp.float32),
        pltpu.VMEM((CHUNK, HID), jnp.float32),
        pltpu.VMEM((NUM_REL, HID), jnp.float32),
        pltpu.VMEM((CHUNK, HID), jnp.float32),
        pltpu.VMEM((CHUNK, HID), jnp.float32),
        pltpu.VMEM_SHARED((NUM_E, HID), jnp.float32),
        pltpu.SemaphoreType.DMA,
        pltpu.SemaphoreType.DMA,
        pltpu.SemaphoreType.DMA,
        pltpu.SemaphoreType.DMA,
        pltpu.SemaphoreType.DMA,
        pltpu.SemaphoreType.DMA,
        pltpu.SemaphoreType.DMA,
        pltpu.SemaphoreType.DMA,
    ],
)
def _sc_conv(emb_hbm, idx2_hbm, wr_hbm, reld_hbm, out_hbm,
             idx2_st, wr_st, rows0, rows1, reld_v, msg0, msg1, agg_sh,
             stsem0, stsem1, stsem2, stsem3, gsem0, gsem1, ssem0, ssem1):
    c = lax.axis_index("c")
    s = lax.axis_index("s")
    tile = c * NS + s
    stsems = [stsem0, stsem1, stsem2, stsem3]
    rows = [rows0, rows1]
    msgs = [msg0, msg1]
    gsems = [gsem0, gsem1]
    ssems = [ssem0, ssem1]

    pltpu.sync_copy(reld_hbm, reld_v)

    @plsc.parallel_loop(0, CHUNK, unroll=4)
    def _zero_row(r):
        for j in range(VPR):
            msg0[r, pl.ds(LANES * j, LANES)] = jnp.zeros((LANES,), jnp.float32)

    for i in range(BLOCKS_PER_TILE):
        blk = s + NS * i

        @pl.when(blk < N_ROW_BLOCKS)
        def _():
            off = pl.multiple_of(blk * ZBLK, 8)
            pltpu.sync_copy(msg0, agg_sh.at[pl.ds(off, ZBLK)])

    plsc.subcore_barrier()

    def _start_strips(ci, slot):
        pltpu.async_copy(idx2_hbm.at[tile, ci], idx2_st.at[slot], stsems[slot])
        pltpu.async_copy(wr_hbm.at[tile, ci], wr_st.at[slot], stsems[slot])

    def _wait_strips(ci, slot):
        pltpu.make_async_copy(idx2_hbm.at[tile, ci], idx2_st.at[slot],
                              stsems[slot]).wait()
        pltpu.make_async_copy(wr_hbm.at[tile, ci], wr_st.at[slot],
                              stsems[slot]).wait()

    def _start_gather(slot, b):
        pltpu.async_copy(emb_hbm.at[idx2_st.at[slot, 0]], rows[b], gsems[b])

    def _wait_gather(slot, b):
        pltpu.make_async_copy(emb_hbm.at[idx2_st.at[slot, 0]], rows[b],
                              gsems[b]).wait()

    def _start_scatter(slot, b):
        pltpu.async_copy(msgs[b], agg_sh.at[idx2_st.at[slot, 1]], ssems[b],
                         add=True)

    def _wait_scatter(slot, b):
        pltpu.make_async_copy(msgs[b], agg_sh.at[idx2_st.at[slot, 1]],
                              ssems[b]).wait()

    def _compute(slot, b):
        rows_v, msg_v = rows[b], msgs[b]
        kvec = jnp.zeros((LANES,), jnp.int32) + slot

        @plsc.parallel_loop(0, CHUNK, unroll=4)
        def _row(e):
            evec = jnp.zeros((LANES,), jnp.int32) + e
            wrv = plsc.load_gather(wr_st, [kvec, evec])
            rb = (wrv * 0.5).astype(jnp.int32)
            wb = wrv - 2.0 * rb.astype(jnp.float32)
            for j in range(VPR):
                col = jnp.arange(LANES, dtype=jnp.int32) + (LANES * j)
                ep = rows_v[e, pl.ds(LANES * j, LANES)]
                rp = plsc.load_gather(reld_v, [rb, col])
                msg_v[e, pl.ds(LANES * j, LANES)] = ep * rp * wb

    def _chunk_step(ci, slot, b):
        nslot = (slot + 1) % NSLOT

        @pl.when(ci + 1 < N_CHUNKS)
        def _():
            _wait_strips(ci + 1, nslot)
            _start_gather(nslot, 1 - b)

        @pl.when(ci >= 2)
        def _():
            _wait_scatter((slot + 2) % NSLOT, b)

        @pl.when(ci + 2 < N_CHUNKS)
        def _():
            _start_strips(ci + 2, (slot + 2) % NSLOT)

        _wait_gather(slot, b)
        _compute(slot, b)
        _start_scatter(slot, b)

    _start_strips(0, 0)
    _start_strips(1, 1)
    _wait_strips(0, 0)
    _start_gather(0, 0)

    def _quad(t, _):
        ci = 4 * t
        _chunk_step(ci, 0, 0)
        _chunk_step(ci + 1, 1, 1)
        _chunk_step(ci + 2, 2, 0)
        _chunk_step(ci + 3, 3, 1)
        return 0

    lax.fori_loop(0, N_CHUNKS // 4, _quad, 0)
    _chunk_step(N_CHUNKS - 2, 0, 0)
    _chunk_step(N_CHUNKS - 1, 1, 1)
    _wait_scatter(0, 0)
    _wait_scatter(1, 1)
    plsc.subcore_barrier()

    for i in range(BLOCKS_PER_TILE):
        blk = s + NS * i

        @pl.when(blk < N_ROW_BLOCKS)
        def _():
            off = pl.multiple_of(blk * ZBLK, 8)
            pltpu.sync_copy(agg_sh.at[pl.ds(off, ZBLK)], msg0)
            pltpu.sync_copy(msg0, out_hbm.at[c, pl.ds(off, ZBLK)])


_TC_BLOCK = 1000


def _tc_finish(emb_ref, p0_ref, p1_ref, w_ref, b_ref, res_ref, out_ref, tmp_ref):
    agg = p0_ref[...] + p1_ref[...]
    h = jnp.tanh(jnp.dot(agg, w_ref[...], preferred_element_type=jnp.float32)
                 + b_ref[...])
    t = res_ref[0, 0] * h
    tmp_ref[...] = t
    out_ref[...] = emb_ref[...] + t


def _tc_call(emb, p0, p1, W, b2, res2):
    grid = (NUM_E // _TC_BLOCK,)
    row_spec = pl.BlockSpec((_TC_BLOCK, HID), lambda i: (i, 0))
    full_spec = pl.BlockSpec((HID, HID), lambda i: (0, 0))
    b_spec = pl.BlockSpec((1, HID), lambda i: (0, 0))
    r_spec = pl.BlockSpec((1, 1), lambda i: (0, 0))
    return pl.pallas_call(
        _tc_finish,
        grid=grid,
        in_specs=[row_spec, row_spec, row_spec, full_spec, b_spec, r_spec],
        out_specs=[row_spec, row_spec],
        out_shape=[
            jax.ShapeDtypeStruct((NUM_E, HID), jnp.float32),
            jax.ShapeDtypeStruct((NUM_E, HID), jnp.float32),
        ],
    )(emb, p0, p1, W, b2, res2)


def kernel(emb, edge_id_his, edge_w_his, rel_his, W, b, rel_diag, res):
    src3 = edge_id_his[0].reshape(NT, N_CHUNKS, 1, CHUNK)
    dst3 = edge_id_his[1].reshape(NT, N_CHUNKS, 1, CHUNK)
    idx2 = jnp.concatenate([src3, dst3], axis=2)
    wr3 = (edge_w_his + 2.0 * rel_his.astype(jnp.float32)
           ).reshape(NT, N_CHUNKS, CHUNK)
    partials = _sc_conv(emb, idx2, wr3, rel_diag)
    out, tmp = _tc_call(emb, partials[0], partials[1], W,
                        b.reshape(1, HID), res.reshape(1, 1))
    return (out, tmp)


# speedup vs baseline: 1.0730x; 1.0730x over previous
import functools

import jax
import jax.numpy as jnp
from jax import lax
from jax.experimental import pallas as pl
from jax.experimental.pallas import tpu as pltpu
from jax.experimental.pallas import tpu_sc as plsc

HID = 128
NUM_E = 10000
NUM_EDGES = 320000
NUM_REL = 200

NC = 2
NS = 16
NT = NC * NS
E_PER_TILE = NUM_EDGES // NT
CHUNK = 40
N_CHUNKS = E_PER_TILE // CHUNK
NSLOT = 4
ZBLK = 40
N_ROW_BLOCKS = NUM_E // ZBLK
BLOCKS_PER_TILE = (N_ROW_BLOCKS + NS - 1) // NS
LANES = 16
VPR = HID // LANES


@functools.partial(
    pl.kernel,
    out_type=jax.ShapeDtypeStruct((NC, NUM_E, HID), jnp.float32),
    mesh=plsc.VectorSubcoreMesh(core_axis_name="c", subcore_axis_name="s"),
    compiler_params=pltpu.CompilerParams(needs_layout_passes=False),
    scratch_types=[
        pltpu.VMEM((NSLOT, CHUNK), jnp.int32),
        pltpu.VMEM((NSLOT, CHUNK), jnp.int32),
        pltpu.VMEM((NSLOT, CHUNK), jnp.int32),
        pltpu.VMEM((NSLOT, CHUNK), jnp.float32),
        pltpu.VMEM((CHUNK, HID), jnp.float32),
        pltpu.VMEM((CHUNK, HID), jnp.float32),
        pltpu.VMEM((NUM_REL, HID), jnp.float32),
        pltpu.VMEM((CHUNK, HID), jnp.float32),
        pltpu.VMEM((CHUNK, HID), jnp.float32),
        pltpu.VMEM_SHARED((NUM_E, HID), jnp.float32),
        pltpu.SemaphoreType.DMA,
        pltpu.SemaphoreType.DMA,
        pltpu.SemaphoreType.DMA,
        pltpu.SemaphoreType.DMA,
        pltpu.SemaphoreType.DMA,
        pltpu.SemaphoreType.DMA,
        pltpu.SemaphoreType.DMA,
        pltpu.SemaphoreType.DMA,
    ],
)
def _sc_conv(emb_hbm, src_hbm, dst_hbm, rel_hbm, w_hbm, reld_hbm, out_hbm,
             src_st, dst_st, rel_st, w_st, rows0, rows1, reld_v, msg0, msg1, agg_sh,
             stsem0, stsem1, stsem2, stsem3, gsem0, gsem1, ssem0, ssem1):
    c = lax.axis_index("c")
    s = lax.axis_index("s")
    tile = c * NS + s
    stsems = [stsem0, stsem1, stsem2, stsem3]
    rows = [rows0, rows1]
    msgs = [msg0, msg1]
    gsems = [gsem0, gsem1]
    ssems = [ssem0, ssem1]

    pltpu.sync_copy(reld_hbm, reld_v)

    @plsc.parallel_loop(0, CHUNK, unroll=4)
    def _zero_row(r):
        for j in range(VPR):
            msg0[r, pl.ds(LANES * j, LANES)] = jnp.zeros((LANES,), jnp.float32)

    for i in range(BLOCKS_PER_TILE):
        blk = s + NS * i

        @pl.when(blk < N_ROW_BLOCKS)
        def _():
            off = pl.multiple_of(blk * ZBLK, 8)
            pltpu.sync_copy(msg0, agg_sh.at[pl.ds(off, ZBLK)])

    plsc.subcore_barrier()

    def _start_strips(ci, slot):
        pltpu.async_copy(src_hbm.at[tile, ci], src_st.at[slot], stsems[slot])
        pltpu.async_copy(dst_hbm.at[tile, ci], dst_st.at[slot], stsems[slot])
        pltpu.async_copy(rel_hbm.at[tile, ci], rel_st.at[slot], stsems[slot])
        pltpu.async_copy(w_hbm.at[tile, ci], w_st.at[slot], stsems[slot])

    def _wait_strips(ci, slot):
        pltpu.make_async_copy(src_hbm.at[tile, ci], src_st.at[slot],
                              stsems[slot]).wait()
        pltpu.make_async_copy(dst_hbm.at[tile, ci], dst_st.at[slot],
                              stsems[slot]).wait()
        pltpu.make_async_copy(rel_hbm.at[tile, ci], rel_st.at[slot],
                              stsems[slot]).wait()
        pltpu.make_async_copy(w_hbm.at[tile, ci], w_st.at[slot],
                              stsems[slot]).wait()

    def _start_gather(slot, b):
        pltpu.async_copy(emb_hbm.at[src_st.at[slot]], rows[b], gsems[b])

    def _wait_gather(slot, b):
        pltpu.make_async_copy(emb_hbm.at[src_st.at[slot]], rows[b],
                              gsems[b]).wait()

    def _start_scatter(slot, b):
        pltpu.async_copy(msgs[b], agg_sh.at[dst_st.at[slot]], ssems[b],
                         add=True)

    def _wait_scatter(slot, b):
        pltpu.make_async_copy(msgs[b], agg_sh.at[dst_st.at[slot]],
                              ssems[b]).wait()

    def _compute(slot, b):
        rows_v, msg_v = rows[b], msgs[b]
        kvec = jnp.zeros((LANES,), jnp.int32) + slot

        @plsc.parallel_loop(0, CHUNK, unroll=4)
        def _row(e):
            evec = jnp.zeros((LANES,), jnp.int32) + e
            wb = plsc.load_gather(w_st, [kvec, evec])
            rb = plsc.load_gather(rel_st, [kvec, evec])
            for j in range(VPR):
                col = jnp.arange(LANES, dtype=jnp.int32) + (LANES * j)
                ep = rows_v[e, pl.ds(LANES * j, LANES)]
                rp = plsc.load_gather(reld_v, [rb, col])
                msg_v[e, pl.ds(LANES * j, LANES)] = ep * rp * wb

    def _chunk_step(ci, slot, b):
        nslot = (slot + 1) % NSLOT

        @pl.when(ci + 1 < N_CHUNKS)
        def _():
            _wait_strips(ci + 1, nslot)
            _start_gather(nslot, 1 - b)

        @pl.when(ci >= 2)
        def _():
            _wait_scatter((slot + 2) % NSLOT, b)

        @pl.when(ci + 2 < N_CHUNKS)
        def _():
            _start_strips(ci + 2, (slot + 2) % NSLOT)

        _wait_gather(slot, b)
        _compute(slot, b)
        _start_scatter(slot, b)

    _start_strips(0, 0)
    _start_strips(1, 1)
    _wait_strips(0, 0)
    _start_gather(0, 0)

    def _quad(t, _):
        ci = 4 * t
        _chunk_step(ci, 0, 0)
        _chunk_step(ci + 1, 1, 1)
        _chunk_step(ci + 2, 2, 0)
        _chunk_step(ci + 3, 3, 1)
        return 0

    lax.fori_loop(0, N_CHUNKS // 4, _quad, 0)
    _chunk_step(N_CHUNKS - 2, 0, 0)
    _chunk_step(N_CHUNKS - 1, 1, 1)
    _wait_scatter(0, 0)
    _wait_scatter(1, 1)
    plsc.subcore_barrier()

    for i in range(BLOCKS_PER_TILE):
        blk = s + NS * i

        @pl.when(blk < N_ROW_BLOCKS)
        def _():
            off = pl.multiple_of(blk * ZBLK, 8)
            pltpu.sync_copy(agg_sh.at[pl.ds(off, ZBLK)], msg0)
            pltpu.sync_copy(msg0, out_hbm.at[c, pl.ds(off, ZBLK)])


_TC_BLOCK = 1000


def _tc_finish(emb_ref, p0_ref, p1_ref, w_ref, b_ref, res_ref, out_ref, tmp_ref):
    agg = p0_ref[...] + p1_ref[...]
    h = jnp.tanh(jnp.dot(agg, w_ref[...], preferred_element_type=jnp.float32)
                 + b_ref[...])
    t = res_ref[0, 0] * h
    tmp_ref[...] = t
    out_ref[...] = emb_ref[...] + t


def _tc_call(emb, p0, p1, W, b2, res2):
    grid = (NUM_E // _TC_BLOCK,)
    row_spec = pl.BlockSpec((_TC_BLOCK, HID), lambda i: (i, 0))
    full_spec = pl.BlockSpec((HID, HID), lambda i: (0, 0))
    b_spec = pl.BlockSpec((1, HID), lambda i: (0, 0))
    r_spec = pl.BlockSpec((1, 1), lambda i: (0, 0))
    return pl.pallas_call(
        _tc_finish,
        grid=grid,
        in_specs=[row_spec, row_spec, row_spec, full_spec, b_spec, r_spec],
        out_specs=[row_spec, row_spec],
        out_shape=[
            jax.ShapeDtypeStruct((NUM_E, HID), jnp.float32),
            jax.ShapeDtypeStruct((NUM_E, HID), jnp.float32),
        ],
    )(emb, p0, p1, W, b2, res2)


def kernel(emb, edge_id_his, edge_w_his, rel_his, W, b, rel_diag, res):
    src3 = edge_id_his[0].reshape(NT, N_CHUNKS, CHUNK)
    dst3 = edge_id_his[1].reshape(NT, N_CHUNKS, CHUNK)
    rel3 = rel_his.reshape(NT, N_CHUNKS, CHUNK)
    w3 = edge_w_his.reshape(NT, N_CHUNKS, CHUNK)
    partials = _sc_conv(emb, src3, dst3, rel3, w3, rel_diag)
    out, tmp = _tc_call(emb, partials[0], partials[1], W,
                        b.reshape(1, HID), res.reshape(1, 1))
    return (out, tmp)
